# Initial kernel scaffold; baseline (speedup 1.0000x reference)
import jax
import jax.numpy as jnp
from jax.experimental import pallas as pl


def kernel(input_ids, emb0, emb1, factor1):
    raise NotImplementedError("write your pallas kernel here")


# speedup vs baseline: 1.2214x; 1.2214x over previous
import functools

import jax
import jax.numpy as jnp
from jax import lax
from jax.experimental import pallas as pl
from jax.experimental.pallas import tpu as pltpu
from jax.experimental.pallas import tpu_sc as plsc

HIDDEN = 64
D0 = 64
D1 = 16
LO1 = 100000
NC = 2
NS = 16
LANES = 16
NW = NC * NS
CH = 128


def _sc_gather(ids, emb0, emb1):
    n = ids.shape[0]
    per_w = n // NW
    n_ch = per_w // CH
    mesh = plsc.VectorSubcoreMesh(
        core_axis_name="c", subcore_axis_name="s",
        num_cores=NC, num_subcores=NS)

    @functools.partial(
        pl.kernel,
        out_type=(
            jax.ShapeDtypeStruct((n, D0), jnp.float32),
            jax.ShapeDtypeStruct((n, D1), jnp.float32),
        ),
        mesh=mesh,
        compiler_params=pltpu.CompilerParams(use_tc_tiling_on_sc=False),
        scratch_types=[
            pltpu.VMEM((per_w,), jnp.int32),
            pltpu.VMEM((per_w,), jnp.int32),
            pltpu.VMEM((per_w,), jnp.int32),
            pltpu.VMEM((CH, D0), jnp.float32),
            pltpu.VMEM((CH, D1), jnp.float32),
            pltpu.SemaphoreType.DMA,
            pltpu.SemaphoreType.DMA,
        ],
    )
    def body(ids_hbm, emb0_hbm, emb1_hbm, rows0_hbm, rows1_hbm,
             ids_v, idx0_v, idx1_v, r0, r1, sem0, sem1):
        wid = lax.axis_index("s") * NC + lax.axis_index("c")
        base = wid * per_w
        pltpu.sync_copy(ids_hbm.at[pl.ds(base, per_w)], ids_v)

        def idx_body(i, carry):
            v = ids_v[pl.ds(i * LANES, LANES)]
            m = v < LO1
            idx0_v[pl.ds(i * LANES, LANES)] = jnp.where(m, v, 0)
            idx1_v[pl.ds(i * LANES, LANES)] = jnp.where(m, 0, v - LO1)
            return carry

        lax.fori_loop(0, per_w // LANES, idx_body, 0)

        def ch_body(c, carry):
            off = c * CH
            g0 = pltpu.async_copy(
                emb0_hbm.at[idx0_v.at[pl.ds(off, CH)]], r0, sem0)
            g1 = pltpu.async_copy(
                emb1_hbm.at[idx1_v.at[pl.ds(off, CH)]], r1, sem1)
            g0.wait()
            g1.wait()
            pltpu.sync_copy(r0, rows0_hbm.at[pl.ds(base + off, CH)])
            pltpu.sync_copy(r1, rows1_hbm.at[pl.ds(base + off, CH)])
            return carry

        lax.fori_loop(0, n_ch, ch_body, 0)

    return body(ids, emb0, emb1)


def _tc_combine(rows0, rows1, factor1):
    n = rows0.shape[0]
    bt = 512

    def body(r0_ref, r1_ref, f_ref, o_ref):
        o_ref[...] = r0_ref[...] + jnp.dot(
            r1_ref[...], f_ref[...], preferred_element_type=jnp.float32)

    return pl.pallas_call(
        body,
        grid=(n // bt,),
        in_specs=[
            pl.BlockSpec((bt, D0), lambda i: (i, 0)),
            pl.BlockSpec((bt, D1), lambda i: (i, 0)),
            pl.BlockSpec((D1, HIDDEN), lambda i: (0, 0)),
        ],
        out_specs=pl.BlockSpec((bt, HIDDEN), lambda i: (i, 0)),
        out_shape=jax.ShapeDtypeStruct((n, HIDDEN), jnp.float32),
    )(rows0, rows1, factor1)


def kernel(input_ids, emb0, emb1, factor1):
    ids = input_ids.reshape(-1).astype(jnp.int32)
    rows0, rows1 = _sc_gather(ids, emb0, emb1)
    out = _tc_combine(rows0, rows1, factor1)
    return out.reshape(input_ids.shape + (HIDDEN,))
